# Initial kernel scaffold; baseline (speedup 1.0000x reference)
import jax
import jax.numpy as jnp
from jax.experimental import pallas as pl


def kernel(q_len, k_len, bias_table):
    raise NotImplementedError("write your pallas kernel here")


# speedup vs baseline: 42.0672x; 42.0672x over previous
import functools

import jax
import jax.numpy as jnp
from jax import lax
from jax.experimental import pallas as pl
from jax.experimental.pallas import tpu as pltpu
from jax.experimental.pallas import tpu_sc as plsc

MAX_DIST = 512
HEADS = 16
SEQ = 2048
TBL = 2 * MAX_DIST + 1
TBL_PAD = 1040
BLK = 16
DW = 4112
N_BLOCKS = SEQ // BLK // 2


def _sc_body(bt_hbm, out_hbm, t_ref, d16_ref):
    wid = lax.axis_index("s") * 2 + lax.axis_index("c")
    h = wid // 2
    half = wid % 2

    pltpu.sync_copy(bt_hbm.at[h], t_ref)

    lane = lax.iota(jnp.int32, 16)
    for b in range(BLK):
        def build(cx, _, b=b):
            x0 = cx * 16
            idx = jnp.clip((2544 + b - x0) - lane, 0, TBL - 1)
            d16_ref[b, pl.ds(x0, 16)] = plsc.load_gather(t_ref, [idx])
            return 0
        lax.fori_loop(0, DW // 16, build, 0, unroll=4)

    ibase = half * (SEQ // 2)

    def emit(blk, _):
        i0 = ibase + blk * BLK
        sb = (SEQ - BLK) - i0
        pltpu.sync_copy(
            d16_ref.at[:, pl.ds(sb, SEQ)],
            out_hbm.at[h, pl.ds(i0, BLK), :],
        )
        return 0

    lax.fori_loop(0, N_BLOCKS, emit, 0)


@functools.partial(
    pl.kernel,
    out_type=jax.ShapeDtypeStruct((HEADS, SEQ, SEQ), jnp.float32),
    mesh=plsc.VectorSubcoreMesh(core_axis_name="c", subcore_axis_name="s"),
    scratch_types=[
        pltpu.VMEM((TBL_PAD,), jnp.float32),
        pltpu.VMEM((BLK, DW), jnp.float32),
    ],
    compiler_params=pltpu.CompilerParams(
        use_tc_tiling_on_sc=False, needs_layout_passes=False
    ),
)
def _rel_pos_bias_sc(bt_hbm, out_hbm, t_ref, d16_ref):
    _sc_body(bt_hbm, out_hbm, t_ref, d16_ref)


def kernel(q_len, k_len, bias_table):
    bt = jnp.pad(bias_table.T, ((0, 0), (0, TBL_PAD - TBL)))
    return _rel_pos_bias_sc(bt)


# speedup vs baseline: 42.2070x; 1.0033x over previous
import functools

import jax
import jax.numpy as jnp
from jax import lax
from jax.experimental import pallas as pl
from jax.experimental.pallas import tpu as pltpu
from jax.experimental.pallas import tpu_sc as plsc

MAX_DIST = 512
HEADS = 16
SEQ = 2048
TBL = 2 * MAX_DIST + 1
TBL_PAD = 1040
BLK = 16
DW = 3056
N_BLOCKS = SEQ // BLK // 2
PIPE = 4


def _sc_body(bt_hbm, out_hbm, t_ref, d16_ref, sem):
    wid = lax.axis_index("s") * 2 + lax.axis_index("c")
    h = wid // 2
    half = wid % 2
    ibase = half * (SEQ // 2)
    xbase = (1 - half) * (SEQ // 2)

    pltpu.sync_copy(bt_hbm.at[h], t_ref)

    lane = lax.iota(jnp.int32, 16)
    for b in range(BLK):
        cb = (2544 + b) - xbase
        def build(cx, _, cb=cb):
            x0 = cx * 16
            idx = jnp.clip((cb - x0) - lane, 0, TBL - 1)
            d16_ref[b, pl.ds(x0, 16)] = plsc.load_gather(t_ref, [idx])
            return 0
        lax.fori_loop(0, DW // 16, build, 0, unroll=4)

    def issue(blk):
        sb = (SEQ // 2 - BLK * 2 + BLK) - BLK * blk
        pltpu.async_copy(
            d16_ref.at[:, pl.ds(sb, SEQ)],
            out_hbm.at[h, pl.ds(ibase + blk * BLK, BLK), :],
            sem,
        )

    def drain_one():
        pltpu.make_async_copy(
            d16_ref.at[:, pl.ds(0, SEQ)],
            out_hbm.at[0, pl.ds(0, BLK), :],
            sem,
        ).wait()

    for blk in range(PIPE):
        issue(blk)

    def emit(blk, _):
        issue(blk)
        drain_one()
        return 0

    lax.fori_loop(PIPE, N_BLOCKS, emit, 0)
    for _ in range(PIPE):
        drain_one()


@functools.partial(
    pl.kernel,
    out_type=jax.ShapeDtypeStruct((HEADS, SEQ, SEQ), jnp.float32),
    mesh=plsc.VectorSubcoreMesh(core_axis_name="c", subcore_axis_name="s"),
    scratch_types=[
        pltpu.VMEM((TBL_PAD,), jnp.float32),
        pltpu.VMEM((BLK, DW), jnp.float32),
        pltpu.SemaphoreType.DMA,
    ],
    compiler_params=pltpu.CompilerParams(
        use_tc_tiling_on_sc=False, needs_layout_passes=False
    ),
)
def _rel_pos_bias_sc(bt_hbm, out_hbm, t_ref, d16_ref, sem):
    _sc_body(bt_hbm, out_hbm, t_ref, d16_ref, sem)


def kernel(q_len, k_len, bias_table):
    bt = jnp.pad(bias_table.T, ((0, 0), (0, TBL_PAD - TBL)))
    return _rel_pos_bias_sc(bt)


# speedup vs baseline: 117.3814x; 2.7811x over previous
import functools

import jax
import jax.numpy as jnp
from jax import lax
from jax.experimental import pallas as pl
from jax.experimental.pallas import tpu as pltpu
from jax.experimental.pallas import tpu_sc as plsc

MAX_DIST = 512
HEADS = 16
SEQ = 2048
TBL = 2 * MAX_DIST + 1
TROW = 1152
BLK = 16
DW = 2944
N_PASS = 8
N_K = 8


def _sc_body(bt_hbm, out_hbm, t_ref, d_refs, sem):
    wid = lax.axis_index("s") * 2 + lax.axis_index("c")
    h = wid // 2
    half = wid % 2
    ibase = half * (SEQ // 2)
    xbase = (1 - half) * (SEQ // 2)

    pltpu.sync_copy(bt_hbm.at[pl.ds(h * TROW, TROW)], t_ref)

    lane = lax.iota(jnp.int32, 16)

    def drain_one():
        pltpu.make_async_copy(
            d_refs[0].at[:, pl.ds(0, SEQ)],
            out_hbm.at[0, pl.ds(0, BLK), :],
            sem,
        ).wait()

    for p in range(N_PASS):
        d_ref = d_refs[p % 2]
        if p >= 2:
            for _ in range(N_K):
                drain_one()
        for b in range(BLK):
            cb = (2544 + b - 16 * p) - xbase
            def build(cx, _, cb=cb, d_ref=d_ref, b=b):
                x0 = cx * 16
                idx = jnp.clip((cb - x0) - lane, 0, TBL - 1)
                d_ref[b, pl.ds(x0, 16)] = plsc.load_gather(t_ref, [idx])
                return 0
            lax.fori_loop(0, DW // 16, build, 0, unroll=4)
        for k in range(N_K):
            blk = (7 - p) % 8 + 8 * k
            off = 896 - 128 * k
            pltpu.async_copy(
                d_ref.at[:, pl.ds(off, SEQ)],
                out_hbm.at[h, pl.ds(ibase + blk * BLK, BLK), :],
                sem,
            )
    for _ in range(2 * N_K):
        drain_one()


@functools.partial(
    pl.kernel,
    out_type=jax.ShapeDtypeStruct((HEADS, SEQ, SEQ), jnp.float32),
    mesh=plsc.VectorSubcoreMesh(core_axis_name="c", subcore_axis_name="s"),
    scratch_types=[
        pltpu.VMEM((TROW,), jnp.float32),
        pltpu.VMEM((BLK, DW), jnp.float32),
        pltpu.VMEM((BLK, DW), jnp.float32),
        pltpu.SemaphoreType.DMA,
    ],
    compiler_params=pltpu.CompilerParams(needs_layout_passes=False),
)
def _rel_pos_bias_sc(bt_hbm, out_hbm, t_ref, d_a, d_b, sem):
    _sc_body(bt_hbm, out_hbm, t_ref, (d_a, d_b), sem)


def kernel(q_len, k_len, bias_table):
    bt = jnp.pad(bias_table.T, ((0, 0), (0, TROW - TBL))).reshape(-1)
    return _rel_pos_bias_sc(bt)


# speedup vs baseline: 122.7789x; 1.0460x over previous
import functools

import jax
import jax.numpy as jnp
from jax import lax
from jax.experimental import pallas as pl
from jax.experimental.pallas import tpu as pltpu
from jax.experimental.pallas import tpu_sc as plsc

MAX_DIST = 512
HEADS = 16
SEQ = 2048
TBL = 2 * MAX_DIST + 1
TROW = 1152
BLK = 16
DW = 2944
N_PASS = 8
N_K = 8


def _sc_body(bt_hbm, out_hbm, t_ref, d_refs, sem):
    wid = lax.axis_index("s") * 2 + lax.axis_index("c")
    h = wid // 2
    half = wid % 2
    ibase = half * (SEQ // 2)
    xbase = (1 - half) * (SEQ // 2)

    pltpu.sync_copy(bt_hbm.at[pl.ds(h * TROW, TROW)], t_ref)

    lane = lax.iota(jnp.int32, 16)
    v_hi = plsc.load_gather(t_ref, [jnp.full((16,), TBL - 1, jnp.int32)])
    v_lo = plsc.load_gather(t_ref, [jnp.zeros((16,), jnp.int32)])

    def drain_one():
        pltpu.make_async_copy(
            d_refs[0].at[:, pl.ds(0, SEQ)],
            out_hbm.at[0, pl.ds(0, BLK), :],
            sem,
        ).wait()

    for p in range(N_PASS):
        d_ref = d_refs[p % 2]
        if p >= 2:
            for _ in range(N_K):
                drain_one()

        base_p = (2544 - 16 * p) - xbase

        def row_build(b, _, d_ref=d_ref, base_p=base_p):
            c = base_p + b
            n_lo = (c - (TBL - 1)) // 16
            n_hi = (c + 15) // 16

            def fill_hi(cx, _):
                d_ref[b, pl.ds(cx * 16, 16)] = v_hi
                return 0

            def mid(cx, _):
                x0 = cx * 16
                idx = jnp.clip((c - x0) - lane, 0, TBL - 1)
                d_ref[b, pl.ds(x0, 16)] = plsc.load_gather(t_ref, [idx])
                return 0

            def fill_lo(cx, _):
                d_ref[b, pl.ds(cx * 16, 16)] = v_lo
                return 0

            lax.fori_loop(0, n_lo, fill_hi, 0)
            lax.fori_loop(n_lo, n_hi, mid, 0)
            lax.fori_loop(n_hi, DW // 16, fill_lo, 0)
            return 0

        lax.fori_loop(0, BLK, row_build, 0)
        for k in range(N_K):
            blk = (7 - p) % 8 + 8 * k
            off = 896 - 128 * k
            pltpu.async_copy(
                d_ref.at[:, pl.ds(off, SEQ)],
                out_hbm.at[h, pl.ds(ibase + blk * BLK, BLK), :],
                sem,
            )
    for _ in range(2 * N_K):
        drain_one()


@functools.partial(
    pl.kernel,
    out_type=jax.ShapeDtypeStruct((HEADS, SEQ, SEQ), jnp.float32),
    mesh=plsc.VectorSubcoreMesh(core_axis_name="c", subcore_axis_name="s"),
    scratch_types=[
        pltpu.VMEM((TROW,), jnp.float32),
        pltpu.VMEM((BLK, DW), jnp.float32),
        pltpu.VMEM((BLK, DW), jnp.float32),
        pltpu.SemaphoreType.DMA,
    ],
    compiler_params=pltpu.CompilerParams(needs_layout_passes=False),
)
def _rel_pos_bias_sc(bt_hbm, out_hbm, t_ref, d_a, d_b, sem):
    _sc_body(bt_hbm, out_hbm, t_ref, (d_a, d_b), sem)


def kernel(q_len, k_len, bias_table):
    bt = jnp.pad(bias_table.T, ((0, 0), (0, TROW - TBL))).reshape(-1)
    return _rel_pos_bias_sc(bt)


# speedup vs baseline: 124.5938x; 1.0148x over previous
import functools

import jax
import jax.numpy as jnp
from jax import lax
from jax.experimental import pallas as pl
from jax.experimental.pallas import tpu as pltpu
from jax.experimental.pallas import tpu_sc as plsc

MAX_DIST = 512
HEADS = 16
SEQ = 2048
TBL = 2 * MAX_DIST + 1
TROW = 1152
BLK = 16
DW = 2944
N_PASS = 8
N_K = 8


def _sc_body(bt_hbm, out_hbm, t_ref, d_refs, sem):
    wid = lax.axis_index("s") * 2 + lax.axis_index("c")
    h = wid // 2
    half = wid % 2
    ibase = half * (SEQ // 2)
    xbase = (1 - half) * (SEQ // 2)

    pltpu.sync_copy(bt_hbm.at[pl.ds(h * TROW, TROW)], t_ref)

    lane = lax.iota(jnp.int32, 16)
    v_lo = plsc.load_gather(t_ref, [jnp.zeros((16,), jnp.int32)])

    def drain_one():
        pltpu.make_async_copy(
            d_refs[0].at[:, pl.ds(0, SEQ)],
            out_hbm.at[0, pl.ds(0, BLK), :],
            sem,
        ).wait()

    for p in range(N_PASS):
        d_ref = d_refs[p % 2]
        if p >= 2:
            for _ in range(N_K):
                drain_one()

        base_p = (2544 - 16 * p) - xbase

        for b in range(BLK):
            c = base_p + b

            def bld(cx, _, d_ref=d_ref, b=b, c=c):
                x0 = cx * 16
                idx = jnp.clip((c - x0) - lane, 0, TBL - 1)
                d_ref[b, pl.ds(x0, 16)] = plsc.load_gather(t_ref, [idx])
                return 0

            if p < 2:
                lax.fori_loop(0, DW // 16, bld, 0, unroll=4)
            else:
                n_lo = (c - (TBL - 1)) // 16
                n_hi = (c + 15) // 16
                lax.fori_loop(n_lo, n_hi, bld, 0)
                d_ref[b, pl.ds(n_hi * 16, 16)] = v_lo
                d_ref[b, pl.ds(n_hi * 16 + 16, 16)] = v_lo
        for k in range(N_K):
            blk = (7 - p) % 8 + 8 * k
            off = 896 - 128 * k
            pltpu.async_copy(
                d_ref.at[:, pl.ds(off, SEQ)],
                out_hbm.at[h, pl.ds(ibase + blk * BLK, BLK), :],
                sem,
            )
    for _ in range(2 * N_K):
        drain_one()


@functools.partial(
    pl.kernel,
    out_type=jax.ShapeDtypeStruct((HEADS, SEQ, SEQ), jnp.float32),
    mesh=plsc.VectorSubcoreMesh(core_axis_name="c", subcore_axis_name="s"),
    scratch_types=[
        pltpu.VMEM((TROW,), jnp.float32),
        pltpu.VMEM((BLK, DW), jnp.float32),
        pltpu.VMEM((BLK, DW), jnp.float32),
        pltpu.SemaphoreType.DMA,
    ],
    compiler_params=pltpu.CompilerParams(needs_layout_passes=False),
)
def _rel_pos_bias_sc(bt_hbm, out_hbm, t_ref, d_a, d_b, sem):
    _sc_body(bt_hbm, out_hbm, t_ref, (d_a, d_b), sem)


def kernel(q_len, k_len, bias_table):
    bt = jnp.pad(bias_table.T, ((0, 0), (0, TROW - TBL))).reshape(-1)
    return _rel_pos_bias_sc(bt)
